# baseline (speedup 1.0000x reference)
import jax
import jax.numpy as jnp
from jax import lax
from jax.experimental import pallas as pl
from jax.experimental.pallas import tpu as pltpu
from jax.experimental.pallas import tpu_sc as plsc

_CH = 12
_WS = 160
_XYZ_MIN = -1.0
_XYZ_MAX = 1.0
_ROW = 128
_NCELL = _WS * _WS * _WS

_NC = 2
_NS = 16
_NW = _NC * _NS

_P = 128
_G = _P // 16

_OFF8 = (0, 1, _WS, _WS + 1,
         _WS * _WS, _WS * _WS + 1, _WS * _WS + _WS, _WS * _WS + _WS + 1)


def _make_kernel(npts):
    ppw = npts // _NW
    nchunk = ppw // _P
    assert ppw * _NW == npts and nchunk * _P == ppw

    mesh = plsc.VectorSubcoreMesh(core_axis_name="c", subcore_axis_name="s",
                                  num_cores=_NC, num_subcores=_NS)

    def body(xs_hbm, ys_hbm, zs_hbm, tab_hbm, out_hbm,
             xs_v, ys_v, zs_v, fx_v, fy_v, fz_v, idx_v, rows_v, outb_v,
             xsem0, xsem1, gsem0, gsem1, osem0, osem1):
        xsem = (xsem0, xsem1)
        gsem = (gsem0, gsem1)
        osem = (osem0, osem1)
        wid = lax.axis_index("s") * _NC + lax.axis_index("c")
        base0 = wid * ppw
        iota = lax.iota(jnp.int32, 16)
        half = jnp.float32(1.0 / (_XYZ_MAX - _XYZ_MIN))
        scale = jnp.float32(_WS - 1)

        def xyz_copies(i, s):
            gb = base0 + i * _P
            sl = pl.ds(gb, _P)
            dsl = pl.ds(s * _P, _P)
            return ((xs_hbm.at[sl], xs_v.at[dsl]),
                    (ys_hbm.at[sl], ys_v.at[dsl]),
                    (zs_hbm.at[sl], zs_v.at[dsl]))

        def fire_xyz(i, s):
            @pl.when(i < nchunk)
            def _():
                for src, dst in xyz_copies(i, s):
                    pltpu.async_copy(src, dst, xsem[s])

        def prep(i, s):
            @pl.when(i < nchunk)
            def _():
                for src, dst in xyz_copies(i, s):
                    pltpu.make_async_copy(src, dst, xsem[s]).wait()

                def grp(g, carry):
                    p0 = g * 16
                    q = s * _P + p0
                    xv = xs_v[pl.ds(q, 16)]
                    yv = ys_v[pl.ds(q, 16)]
                    zv = zs_v[pl.ds(q, 16)]
                    px = (xv - _XYZ_MIN) * half * scale
                    py = (yv - _XYZ_MIN) * half * scale
                    pz = (zv - _XYZ_MIN) * half * scale
                    xi = jnp.clip(px.astype(jnp.int32), 0, _WS - 2)
                    yi = jnp.clip(py.astype(jnp.int32), 0, _WS - 2)
                    zi = jnp.clip(pz.astype(jnp.int32), 0, _WS - 2)
                    fx_v[pl.ds(q, 16)] = px - xi.astype(jnp.float32)
                    fy_v[pl.ds(q, 16)] = py - yi.astype(jnp.float32)
                    fz_v[pl.ds(q, 16)] = pz - zi.astype(jnp.float32)
                    idx_v[pl.ds(q, 16)] = xi * (_WS * _WS) + yi * _WS + zi
                    return carry

                lax.fori_loop(0, _G, grp, None)

                pltpu.async_copy(
                    tab_hbm.at[idx_v.at[pl.ds(s * _P, _P)]],
                    rows_v.at[pl.ds(s * _P, _P), :],
                    gsem[s])

        def blendout(i, s):
            gb = base0 + i * _P
            pltpu.make_async_copy(
                tab_hbm.at[idx_v.at[pl.ds(s * _P, _P)]],
                rows_v.at[pl.ds(s * _P, _P), :],
                gsem[s]).wait()
            @pl.when(i >= 2)
            def _():
                pltpu.make_async_copy(outb_v.at[pl.ds(s * _P, _P), :],
                                      out_hbm.at[pl.ds(0, _P), :],
                                      osem[s]).wait()

            def grp(g, carry):
                p0 = g * 16
                q = s * _P + p0
                fx = fx_v[pl.ds(q, 16)]
                fy = fy_v[pl.ds(q, 16)]
                fz = fz_v[pl.ds(q, 16)]
                one = jnp.float32(1.0)
                gx = one - fx
                gy = one - fy
                gz = one - fz
                wa = gx * gy
                wb = gx * fy
                wc = fx * gy
                wd = fx * fy
                w = (wa * gz, wa * fz, wb * gz, wb * fz,
                     wc * gz, wc * fz, wd * gz, wd * fz)
                row = q + iota
                acc = [None] * _CH
                for j in range(8):
                    for c in range(_CH):
                        col = jnp.full((16,), j * _CH + c, jnp.int32)
                        val = plsc.load_gather(rows_v, [row, col])
                        if acc[c] is None:
                            acc[c] = w[j] * val
                        else:
                            acc[c] = acc[c] + w[j] * val
                pv = p0 + iota
                for c in range(_CH):
                    plsc.store_scatter(outb_v, [q + iota, jnp.full((16,), c, jnp.int32)],
                                       acc[c])
                return carry

            lax.fori_loop(0, _G, grp, None)
            pltpu.async_copy(outb_v.at[pl.ds(s * _P, _P), :],
                             out_hbm.at[pl.ds(gb, _P), :], osem[s])

        fire_xyz(0, 0)
        fire_xyz(1, 1)
        prep(0, 0)
        fire_xyz(2, 0)

        def step(i, s):
            prep(i + 1, s ^ 1)
            fire_xyz(i + 3, s ^ 1)
            blendout(i, s)

        def dbl(j, carry):
            i = j * 2
            step(i, 0)
            step(i + 1, 1)
            return carry

        lax.fori_loop(0, nchunk // 2, dbl, None)

        for s in (0, 1):
            pltpu.make_async_copy(outb_v.at[pl.ds(s * _P, _P), :],
                                  out_hbm.at[pl.ds(0, _P), :], osem[s]).wait()

    scratch = [
        pltpu.VMEM((2 * _P,), jnp.float32),
        pltpu.VMEM((2 * _P,), jnp.float32),
        pltpu.VMEM((2 * _P,), jnp.float32),
        pltpu.VMEM((2 * _P,), jnp.float32),
        pltpu.VMEM((2 * _P,), jnp.float32),
        pltpu.VMEM((2 * _P,), jnp.float32),
        pltpu.VMEM((2 * _P,), jnp.int32),
        pltpu.VMEM((2 * _P, _ROW), jnp.float32),
        pltpu.VMEM((2 * _P, _CH), jnp.float32),
        pltpu.SemaphoreType.DMA,
        pltpu.SemaphoreType.DMA,
        pltpu.SemaphoreType.DMA,
        pltpu.SemaphoreType.DMA,
        pltpu.SemaphoreType.DMA,
        pltpu.SemaphoreType.DMA,
    ]
    return pl.kernel(body,
                     out_type=jax.ShapeDtypeStruct((npts, _CH), jnp.float32),
                     mesh=mesh,
                     compiler_params=pltpu.CompilerParams(
                         needs_layout_passes=False),
                     scratch_types=scratch)


def kernel(xyz, grid):
    npts = xyz.shape[0]
    xs = xyz[:, 0]
    ys = xyz[:, 1]
    zs = xyz[:, 2]
    a = grid.reshape(_NCELL, _CH)
    parts = [a if off == 0 else jnp.concatenate([a[off:], a[:off]], axis=0)
             for off in _OFF8]
    tab = jnp.pad(jnp.concatenate(parts, axis=1),
                  ((0, 0), (0, _ROW - 8 * _CH)))
    return _make_kernel(npts)(xs, ys, zs, tab)


# speedup vs baseline: 6.7033x; 6.7033x over previous
import jax
import jax.numpy as jnp
from jax import lax
from jax.experimental import pallas as pl
from jax.experimental.pallas import tpu as pltpu
from jax.experimental.pallas import tpu_sc as plsc

_CH = 12
_WS = 160
_XYZ_MIN = -1.0
_XYZ_MAX = 1.0
_ROW = 128
_NCELL = _WS * _WS * _WS

_NC = 2
_NS = 16
_NW = _NC * _NS

_P = 32
_G = _P // 16
_RPC = 8 * _P

_COFF = (0, _WS, _WS * _WS, _WS * _WS + _WS)


def _make_kernel(npts):
    ppw = npts // _NW
    nchunk = ppw // _P
    assert ppw * _NW == npts and nchunk * _P == ppw

    mesh = plsc.VectorSubcoreMesh(core_axis_name="c", subcore_axis_name="s",
                                  num_cores=_NC, num_subcores=_NS)

    def body(xs_hbm, ys_hbm, zs_hbm, tab_hbm, out_hbm,
             xs_v, ys_v, zs_v, fx_v, fy_v, fz_v, idx_v, rows_v, outb_v,
             xsem0, xsem1, gsem0, gsem1, osem0, osem1):
        xsem = (xsem0, xsem1)
        gsem = (gsem0, gsem1)
        osem = (osem0, osem1)
        wid = lax.axis_index("s") * _NC + lax.axis_index("c")
        base0 = wid * ppw
        iota = lax.iota(jnp.int32, 16)
        half = jnp.float32(1.0 / (_XYZ_MAX - _XYZ_MIN))
        scale = jnp.float32(_WS - 1)

        def xyz_copies(i, s):
            gb = base0 + i * _P
            sl = pl.ds(gb, _P)
            dsl = pl.ds(s * _P, _P)
            return ((xs_hbm.at[sl], xs_v.at[dsl]),
                    (ys_hbm.at[sl], ys_v.at[dsl]),
                    (zs_hbm.at[sl], zs_v.at[dsl]))

        def fire_xyz(i, s):
            @pl.when(i < nchunk)
            def _():
                for src, dst in xyz_copies(i, s):
                    pltpu.async_copy(src, dst, xsem[s])

        def prep(i, s):
            @pl.when(i < nchunk)
            def _():
                for src, dst in xyz_copies(i, s):
                    pltpu.make_async_copy(src, dst, xsem[s]).wait()

                def grp(g, carry):
                    p0 = g * 16
                    q = s * _P + p0
                    xv = xs_v[pl.ds(q, 16)]
                    yv = ys_v[pl.ds(q, 16)]
                    zv = zs_v[pl.ds(q, 16)]
                    px = (xv - _XYZ_MIN) * half * scale
                    py = (yv - _XYZ_MIN) * half * scale
                    pz = (zv - _XYZ_MIN) * half * scale
                    xi = jnp.clip(px.astype(jnp.int32), 0, _WS - 2)
                    yi = jnp.clip(py.astype(jnp.int32), 0, _WS - 2)
                    zi = jnp.clip(pz.astype(jnp.int32), 0, _WS - 2)
                    fx_v[pl.ds(q, 16)] = px - xi.astype(jnp.float32)
                    fy_v[pl.ds(q, 16)] = py - yi.astype(jnp.float32)
                    fz_v[pl.ds(q, 16)] = pz - zi.astype(jnp.float32)
                    base = xi * (_WS * _WS) + yi * _WS + zi
                    for k in range(4):
                        r0 = base + _COFF[k]
                        qi = s * _RPC + (2 * k) * _P + p0
                        idx_v[pl.ds(qi, 16)] = r0
                        idx_v[pl.ds(qi + _P, 16)] = r0 + 1
                    return carry

                lax.fori_loop(0, _G, grp, None)

                pltpu.async_copy(
                    tab_hbm.at[idx_v.at[pl.ds(s * _RPC, _RPC)]],
                    rows_v.at[pl.ds(s * _RPC, _RPC), :],
                    gsem[s])

        def blendout(i, s):
            gb = base0 + i * _P
            pltpu.make_async_copy(
                tab_hbm.at[idx_v.at[pl.ds(s * _RPC, _RPC)]],
                rows_v.at[pl.ds(s * _RPC, _RPC), :],
                gsem[s]).wait()
            @pl.when(i >= 2)
            def _():
                pltpu.make_async_copy(outb_v.at[pl.ds(s * _P, _P), :],
                                      out_hbm.at[pl.ds(0, _P), :],
                                      osem[s]).wait()

            def grp(g, carry):
                p0 = g * 16
                q = s * _P + p0
                fx = fx_v[pl.ds(q, 16)]
                fy = fy_v[pl.ds(q, 16)]
                fz = fz_v[pl.ds(q, 16)]
                one = jnp.float32(1.0)
                gx = one - fx
                gy = one - fy
                gz = one - fz
                wa = gx * gy
                wb = gx * fy
                wc = fx * gy
                wd = fx * fy
                w = (wa * gz, wa * fz, wb * gz, wb * fz,
                     wc * gz, wc * fz, wd * gz, wd * fz)
                pv = p0 + iota
                acc = [None] * _CH
                for k in range(4):
                    for dz in range(2):
                        row = s * _RPC + (2 * k + dz) * _P + pv
                        wv = w[2 * k + dz]
                        for c in range(_CH):
                            val = plsc.load_gather(
                                rows_v, [row, jnp.full((16,), c, jnp.int32)])
                            if acc[c] is None:
                                acc[c] = wv * val
                            else:
                                acc[c] = acc[c] + wv * val
                for c in range(_CH):
                    plsc.store_scatter(outb_v, [q + iota, jnp.full((16,), c, jnp.int32)],
                                       acc[c])
                return carry

            lax.fori_loop(0, _G, grp, None)
            pltpu.async_copy(outb_v.at[pl.ds(s * _P, _P), :],
                             out_hbm.at[pl.ds(gb, _P), :], osem[s])

        fire_xyz(0, 0)
        fire_xyz(1, 1)
        prep(0, 0)
        fire_xyz(2, 0)

        def step(i, s):
            prep(i + 1, s ^ 1)
            fire_xyz(i + 3, s ^ 1)
            blendout(i, s)

        def dbl(j, carry):
            i = j * 2
            step(i, 0)
            step(i + 1, 1)
            return carry

        lax.fori_loop(0, nchunk // 2, dbl, None)

        for s in (0, 1):
            pltpu.make_async_copy(outb_v.at[pl.ds(s * _P, _P), :],
                                  out_hbm.at[pl.ds(0, _P), :], osem[s]).wait()

    scratch = [
        pltpu.VMEM((2 * _P,), jnp.float32),
        pltpu.VMEM((2 * _P,), jnp.float32),
        pltpu.VMEM((2 * _P,), jnp.float32),
        pltpu.VMEM((2 * _P,), jnp.float32),
        pltpu.VMEM((2 * _P,), jnp.float32),
        pltpu.VMEM((2 * _P,), jnp.float32),
        pltpu.VMEM((2 * _RPC,), jnp.int32),
        pltpu.VMEM((2 * _RPC, _ROW), jnp.float32),
        pltpu.VMEM((2 * _P, _CH), jnp.float32),
        pltpu.SemaphoreType.DMA,
        pltpu.SemaphoreType.DMA,
        pltpu.SemaphoreType.DMA,
        pltpu.SemaphoreType.DMA,
        pltpu.SemaphoreType.DMA,
        pltpu.SemaphoreType.DMA,
    ]
    return pl.kernel(body,
                     out_type=jax.ShapeDtypeStruct((npts, _CH), jnp.float32),
                     mesh=mesh,
                     compiler_params=pltpu.CompilerParams(
                         needs_layout_passes=False),
                     scratch_types=scratch)


def kernel(xyz, grid):
    npts = xyz.shape[0]
    xs = xyz[:, 0]
    ys = xyz[:, 1]
    zs = xyz[:, 2]
    gridp = jnp.pad(grid, ((0, 0), (0, 0), (0, 0), (0, _ROW - _CH)))
    tab = gridp.reshape(_NCELL, _ROW)
    return _make_kernel(npts)(xs, ys, zs, tab)
